# baseline (speedup 1.0000x reference)
import functools

import jax
import jax.numpy as jnp
from jax.experimental import pallas as pl
from jax.experimental.pallas import tpu as pltpu

D_MODEL = 256
NHEAD = 4
DH = D_MODEL // NHEAD
N_ROWS = 4096
BLOCK_M = 2048
NB = N_ROWS // BLOCK_M
SCALE = 1.0 / (DH ** 0.5)
LOG2E = 1.4426950408889634
BF = jnp.bfloat16


def _ln(x, g, b):
    m = jnp.mean(x, axis=-1, keepdims=True)
    v = jnp.mean((x - m) ** 2, axis=-1, keepdims=True)
    return (x - m) * jax.lax.rsqrt(v + 1e-5) * g + b


def _mm(a, b):
    return jnp.dot(a, b, preferred_element_type=jnp.float32)


def _encoder_kernel(d0_ref, d1_ref, ln0g, ln0b, wq, bq, wk, bk, wv, bv,
                    wm, bm, ln1g, ln1b, w1, b1, w2, b2, ln2g, ln2b,
                    out0_ref, out1_ref, k_s, v_s):
    i = pl.program_id(0)
    d = pl.program_id(1)

    @pl.when(i == 0)
    def _compute_kv():
        src = jnp.where(d == 0, d1_ref[...], d0_ref[...])
        sn = _ln(src.T, ln0g[...], ln0b[...]).astype(BF)
        k_s[d] = (_mm(sn, wk[...]) + bk[...]).astype(BF)
        v_full = _mm(sn, wv[...]) + bv[...]
        ones_col = jnp.ones((N_ROWS, 1), BF)
        pad = jnp.zeros((N_ROWS, 128 - DH - 1), BF)
        v_s[d] = jnp.concatenate(
            [jnp.concatenate(
                [v_full[:, h * DH:(h + 1) * DH].astype(BF), ones_col, pad],
                axis=1) for h in range(NHEAD)], axis=1)

    xq_raw = jnp.where(d == 0,
                       d0_ref[:, pl.ds(i * BLOCK_M, BLOCK_M)],
                       d1_ref[:, pl.ds(i * BLOCK_M, BLOCK_M)])
    xn = _ln(xq_raw.T, ln0g[...], ln0b[...])
    xn16 = xn.astype(BF)
    q = (_mm(xn16, wq[...]) + bq[...]).astype(BF)

    msgs = []
    for h in range(NHEAD):
        qh = q[:, h * DH:(h + 1) * DH]
        kh = k_s[d, :, h * DH:(h + 1) * DH]
        s = jax.lax.dot_general(
            qh, kh, (((1,), (1,)), ((), ())),
            preferred_element_type=jnp.float32)
        e = jax.lax.exp2(s.astype(BF))
        ms = _mm(e, v_s[d, :, h * 128:(h + 1) * 128])
        msgs.append(ms[:, :DH] / ms[:, DH:DH + 1])
    msg = jnp.concatenate(msgs, axis=-1).astype(BF)

    m2 = _ln(_mm(msg, wm[...]) + bm[...], ln1g[...], ln1b[...])
    hcat = jnp.concatenate([xn16, m2.astype(BF)], axis=-1)
    h1 = _mm(hcat, w1[...]) + b1[...]
    hmid = (0.5 * h1 * (1.0 + jax.lax.erf(h1 * (2.0 ** -0.5)))).astype(BF)
    y = _mm(hmid, w2[...]) + b2[...]
    res = xq_raw + _ln(y, ln2g[...], ln2b[...]).T

    @pl.when(d == 0)
    def _():
        out0_ref[...] = res

    @pl.when(d == 1)
    def _():
        out1_ref[...] = res


@functools.partial(jax.jit, static_argnames=())
def kernel(desc0, desc1, kpts0, kpts1, ln0_g, ln0_b, Wq, bq, Wk, bk, Wv, bv,
           Wm, bm, ln1_g, ln1_b, W1, b1, W2, b2, ln2_g, ln2_b):
    del kpts0, kpts1

    def row2(a):
        return a.reshape(1, -1)

    full = lambda shape: pl.BlockSpec(shape, lambda i, d: (0,) * len(shape))
    o0, o1 = pl.pallas_call(
        _encoder_kernel,
        grid=(NB, 2),
        in_specs=[
            full((D_MODEL, N_ROWS)),
            full((D_MODEL, N_ROWS)),
            full((1, D_MODEL)), full((1, D_MODEL)),
            full((D_MODEL, D_MODEL)), full((1, D_MODEL)),
            full((D_MODEL, D_MODEL)), full((1, D_MODEL)),
            full((D_MODEL, D_MODEL)), full((1, D_MODEL)),
            full((D_MODEL, D_MODEL)), full((1, D_MODEL)),
            full((1, D_MODEL)), full((1, D_MODEL)),
            full((2 * D_MODEL, 2 * D_MODEL)), full((1, 2 * D_MODEL)),
            full((2 * D_MODEL, D_MODEL)), full((1, D_MODEL)),
            full((1, D_MODEL)), full((1, D_MODEL)),
        ],
        out_specs=[
            pl.BlockSpec((D_MODEL, BLOCK_M), lambda i, d: (0, i)),
            pl.BlockSpec((D_MODEL, BLOCK_M), lambda i, d: (0, i)),
        ],
        out_shape=[
            jax.ShapeDtypeStruct((D_MODEL, N_ROWS), jnp.float32),
            jax.ShapeDtypeStruct((D_MODEL, N_ROWS), jnp.float32),
        ],
        scratch_shapes=[
            pltpu.VMEM((2, N_ROWS, D_MODEL), BF),
            pltpu.VMEM((2, N_ROWS, NHEAD * 128), BF),
        ],
    )(desc0[0], desc1[0], row2(ln0_g), row2(ln0_b),
      (Wq * (SCALE * LOG2E)).astype(BF), row2(bq * (SCALE * LOG2E)),
      Wk.astype(BF), row2(bk), Wv.astype(BF), row2(bv),
      Wm.astype(BF), row2(bm), row2(ln1_g), row2(ln1_b),
      W1.astype(BF), row2(b1), W2.astype(BF), row2(b2),
      row2(ln2_g), row2(ln2_b))

    return (o0[None], o1[None])


# speedup vs baseline: 1.2903x; 1.2903x over previous
import functools

import jax
import jax.numpy as jnp
from jax.experimental import pallas as pl
from jax.experimental.pallas import tpu as pltpu

D_MODEL = 256
NHEAD = 4
DH = D_MODEL // NHEAD
N_ROWS = 4096
BLOCK_M = 1024
NB = N_ROWS // BLOCK_M
SCALE = 1.0 / (DH ** 0.5)
LOG2E = 1.4426950408889634
BF = jnp.bfloat16


def _ln(x, g, b):
    m = jnp.mean(x, axis=-1, keepdims=True)
    v = jnp.mean((x - m) ** 2, axis=-1, keepdims=True)
    return (x - m) * jax.lax.rsqrt(v + 1e-5) * g + b


def _mm(a, b):
    return jnp.dot(a, b, preferred_element_type=jnp.float32)


def _encoder_kernel(d0_ref, d1_ref, ln0g, ln0b, wq, bq, wk, bk, wv, bv,
                    wm, bm, ln1g, ln1b, w1, b1, w2, b2, ln2g, ln2b,
                    out0_ref, out1_ref, k_s, v_s, xn_s):
    i = pl.program_id(0)
    d = pl.program_id(1)

    @pl.when(jnp.logical_and(i == 0, d == 0))
    def _compute_kv():
        for dd in (0, 1):
            src_ref = d1_ref if dd == 0 else d0_ref
            sn = _ln(src_ref[...].T, ln0g[...], ln0b[...]).astype(BF)
            xn_s[1 - dd] = sn
            k_s[dd] = (_mm(sn, wk[...]) + bk[...]).astype(BF)
            v_full = _mm(sn, wv[...]) + bv[...]
            ones_col = jnp.ones((N_ROWS, 1), BF)
            pad = jnp.zeros((N_ROWS, 128 - DH - 1), BF)
            v_s[dd] = jnp.concatenate(
                [jnp.concatenate(
                    [v_full[:, h * DH:(h + 1) * DH].astype(BF), ones_col,
                     pad], axis=1) for h in range(NHEAD)], axis=1)

    xq_raw = jnp.where(d == 0,
                       d0_ref[:, pl.ds(i * BLOCK_M, BLOCK_M)],
                       d1_ref[:, pl.ds(i * BLOCK_M, BLOCK_M)])
    xn16 = xn_s[d, pl.ds(i * BLOCK_M, BLOCK_M)]
    q = (_mm(xn16, wq[...]) + bq[...]).astype(BF)

    msgs = []
    for h in range(NHEAD):
        qh = q[:, h * DH:(h + 1) * DH]
        kh = k_s[d, :, h * DH:(h + 1) * DH]
        s = jax.lax.dot_general(
            qh, kh, (((1,), (1,)), ((), ())),
            preferred_element_type=jnp.float32)
        e = jax.lax.exp2(s.astype(BF))
        ms = _mm(e, v_s[d, :, h * 128:(h + 1) * 128])
        msgs.append(ms[:, :DH] / ms[:, DH:DH + 1])
    msg = jnp.concatenate(msgs, axis=-1).astype(BF)

    m2 = _ln(_mm(msg, wm[...]) + bm[...], ln1g[...], ln1b[...])
    hcat = jnp.concatenate([xn16, m2.astype(BF)], axis=-1)
    h1 = _mm(hcat, w1[...]) + b1[...]
    hmid = (0.5 * h1 * (1.0 + jax.lax.erf(h1 * (2.0 ** -0.5)))).astype(BF)
    y = _mm(hmid, w2[...]) + b2[...]
    res = xq_raw + _ln(y, ln2g[...], ln2b[...]).T

    @pl.when(d == 0)
    def _():
        out0_ref[...] = res

    @pl.when(d == 1)
    def _():
        out1_ref[...] = res


@functools.partial(jax.jit, static_argnames=())
def kernel(desc0, desc1, kpts0, kpts1, ln0_g, ln0_b, Wq, bq, Wk, bk, Wv, bv,
           Wm, bm, ln1_g, ln1_b, W1, b1, W2, b2, ln2_g, ln2_b):
    del kpts0, kpts1

    def row2(a):
        return a.reshape(1, -1)

    full = lambda shape: pl.BlockSpec(shape, lambda i, d: (0,) * len(shape))
    o0, o1 = pl.pallas_call(
        _encoder_kernel,
        grid=(NB, 2),
        in_specs=[
            full((D_MODEL, N_ROWS)),
            full((D_MODEL, N_ROWS)),
            full((1, D_MODEL)), full((1, D_MODEL)),
            full((D_MODEL, D_MODEL)), full((1, D_MODEL)),
            full((D_MODEL, D_MODEL)), full((1, D_MODEL)),
            full((D_MODEL, D_MODEL)), full((1, D_MODEL)),
            full((D_MODEL, D_MODEL)), full((1, D_MODEL)),
            full((1, D_MODEL)), full((1, D_MODEL)),
            full((2 * D_MODEL, 2 * D_MODEL)), full((1, 2 * D_MODEL)),
            full((2 * D_MODEL, D_MODEL)), full((1, D_MODEL)),
            full((1, D_MODEL)), full((1, D_MODEL)),
        ],
        out_specs=[
            pl.BlockSpec((D_MODEL, BLOCK_M), lambda i, d: (0, i)),
            pl.BlockSpec((D_MODEL, BLOCK_M), lambda i, d: (0, i)),
        ],
        out_shape=[
            jax.ShapeDtypeStruct((D_MODEL, N_ROWS), jnp.float32),
            jax.ShapeDtypeStruct((D_MODEL, N_ROWS), jnp.float32),
        ],
        scratch_shapes=[
            pltpu.VMEM((2, N_ROWS, D_MODEL), BF),
            pltpu.VMEM((2, N_ROWS, NHEAD * 128), BF),
            pltpu.VMEM((2, N_ROWS, D_MODEL), BF),
        ],
    )(desc0[0], desc1[0], row2(ln0_g), row2(ln0_b),
      (Wq * (SCALE * LOG2E)).astype(BF), row2(bq * (SCALE * LOG2E)),
      Wk.astype(BF), row2(bk), Wv.astype(BF), row2(bv),
      Wm.astype(BF), row2(bm), row2(ln1_g), row2(ln1_b),
      W1.astype(BF), row2(b1), W2.astype(BF), row2(b2),
      row2(ln2_g), row2(ln2_b))

    return (o0[None], o1[None])
